# baseline (speedup 1.0000x reference)
import functools

import jax
import jax.numpy as jnp
from jax import lax
from jax.experimental import pallas as pl
from jax.experimental.pallas import tpu as pltpu
from jax.experimental.pallas import tpu_sc as plsc

NB = 64
NF = 1280
NG = 512
NIDS = 65536
LANES = 16
NWORKERS = 32
BPW = NB // NWORKERS


def _sc_mask_body(aggr_hbm, mask_hbm, ids_v, mask_v):
    cid = lax.axis_index("c")
    sid = lax.axis_index("s")
    wid = sid * 2 + cid
    zeros = jnp.zeros((LANES,), jnp.float32)
    ones = jnp.ones((LANES,), jnp.float32)
    for bi in range(BPW):
        b = wid * BPW + bi

        def _zero(i, _):
            mask_v[pl.ds(i * LANES, LANES)] = zeros
            return 0

        lax.fori_loop(0, NF // LANES, _zero, 0, unroll=8)
        pltpu.sync_copy(aggr_hbm.at[b], ids_v)

        def _scatter(i, _):
            idx = ids_v[pl.ds(i * LANES, LANES)]
            plsc.store_scatter(mask_v, [idx], ones)
            return 0

        lax.fori_loop(0, NIDS // LANES, _scatter, 0, unroll=8)
        pltpu.sync_copy(mask_v, mask_hbm.at[b])


@jax.jit
def _sc_mask(aggr_info):
    mesh = plsc.VectorSubcoreMesh(core_axis_name="c", subcore_axis_name="s")
    return pl.kernel(
        _sc_mask_body,
        out_type=jax.ShapeDtypeStruct((NB, NF), jnp.float32),
        mesh=mesh,
        scratch_types=[
            pltpu.VMEM((NIDS,), jnp.int32),
            pltpu.VMEM((NF,), jnp.float32),
        ],
        compiler_params=pltpu.CompilerParams(needs_layout_passes=False),
    )(aggr_info)


def _reduce_body(flow_ref, out_ref):
    s = flow_ref[0, 0:2560, :] + flow_ref[0, 2560:5120, :]
    for half in (1280, 640, 320, 160, 80, 40):
        s = s[0:half] + s[half : 2 * half]
    s = s[0:20] + s[20:40]
    out_ref[0] = s * jnp.float32(2.0 / NG)


@jax.jit
def _tc_reduce(flow3):
    return pl.pallas_call(
        _reduce_body,
        grid=(NB,),
        in_specs=[pl.BlockSpec((1, 5120, 128), lambda b: (b, 0, 0))],
        out_specs=pl.BlockSpec((1, 20, 128), lambda b: (b, 0, 0)),
        out_shape=jax.ShapeDtypeStruct((NB, 20, 128), jnp.float32),
    )(flow3)


def _combine_body(avg_ref, prob_ref, mask_ref, out_ref):
    d = (avg_ref[...] - prob_ref[...]) * mask_ref[...][:, None, :]
    out_ref[0, 0] = jnp.sum(d * d) * jnp.float32(1.0 / (NB * NF * 2))


@jax.jit
def _tc_combine(avg_t, prob_t, mask):
    return pl.pallas_call(
        _combine_body,
        out_specs=pl.BlockSpec(memory_space=pltpu.SMEM),
        out_shape=jax.ShapeDtypeStruct((1, 1), jnp.float32),
    )(avg_t, prob_t, mask)


def kernel(flow, prob, aggr_info):
    flow3 = (
        jnp.transpose(flow, (0, 2, 3, 4, 1))
        .reshape(NB, NG // 2, 2, NF // 128, 128)
        .transpose(0, 1, 3, 2, 4)
        .reshape(NB, 5120, 128)
    )
    prob_t = jnp.transpose(prob, (0, 2, 1))
    mask = _sc_mask(aggr_info)
    avg20 = _tc_reduce(flow3)
    avg_t = (
        avg20.reshape(NB, NF // 128, 2, 128)
        .transpose(0, 2, 1, 3)
        .reshape(NB, 2, NF)
    )
    loss = _tc_combine(avg_t, prob_t, mask)[0, 0]
    avg10 = jnp.transpose(avg_t[0, :, 0:10])
    return (loss, avg10)


# speedup vs baseline: 1.0060x; 1.0060x over previous
import functools

import jax
import jax.numpy as jnp
from jax import lax
from jax.experimental import pallas as pl
from jax.experimental.pallas import tpu as pltpu
from jax.experimental.pallas import tpu_sc as plsc

NB = 64
NF = 1280
NG = 512
NIDS = 65536
LANES = 16
NWORKERS = 32
BPW = NB // NWORKERS


def _sc_mask_body(aggr_hbm, mask_hbm, ids_v, mask_v):
    cid = lax.axis_index("c")
    sid = lax.axis_index("s")
    wid = sid * 2 + cid
    zeros = jnp.zeros((LANES,), jnp.float32)
    ones = jnp.ones((LANES,), jnp.float32)
    for bi in range(BPW):
        b = wid * BPW + bi

        def _zero(i, _):
            mask_v[pl.ds(i * LANES, LANES)] = zeros
            return 0

        lax.fori_loop(0, NF // LANES, _zero, 0, unroll=8)
        pltpu.sync_copy(aggr_hbm.at[b], ids_v)

        def _scatter(i, _):
            idx = ids_v[pl.ds(i * LANES, LANES)]
            plsc.store_scatter(mask_v, [idx], ones)
            return 0

        lax.fori_loop(0, NIDS // LANES, _scatter, 0, unroll=8)
        pltpu.sync_copy(mask_v, mask_hbm.at[b])


@jax.jit
def _sc_mask(aggr_info):
    mesh = plsc.VectorSubcoreMesh(core_axis_name="c", subcore_axis_name="s")
    return pl.kernel(
        _sc_mask_body,
        out_type=jax.ShapeDtypeStruct((NB, NF), jnp.float32),
        mesh=mesh,
        scratch_types=[
            pltpu.VMEM((NIDS,), jnp.int32),
            pltpu.VMEM((NF,), jnp.float32),
        ],
        compiler_params=pltpu.CompilerParams(needs_layout_passes=False),
    )(aggr_info)


def _reduce_body(flow_ref, out_ref):
    s = flow_ref[0, 0:2560, :] + flow_ref[0, 2560:5120, :]
    for half in (1280, 640, 320, 160, 80, 40):
        s = s[0:half] + s[half : 2 * half]
    s = (s[0:20] + s[20:40]) * jnp.float32(2.0 / NG)
    rows = [
        jnp.concatenate([s[t * 2 + c : t * 2 + c + 1, :] for t in range(10)], axis=1)
        for c in range(2)
    ]
    out_ref[0] = jnp.concatenate(rows, axis=0)


@jax.jit
def _tc_reduce(flow3):
    return pl.pallas_call(
        _reduce_body,
        grid=(NB,),
        in_specs=[pl.BlockSpec((1, 5120, 128), lambda b: (b, 0, 0))],
        out_specs=pl.BlockSpec((1, 2, NF), lambda b: (b, 0, 0)),
        out_shape=jax.ShapeDtypeStruct((NB, 2, NF), jnp.float32),
    )(flow3)


def _combine_body(avg_ref, prob_ref, mask_ref, out_ref):
    d = (avg_ref[...] - prob_ref[...]) * mask_ref[...][:, None, :]
    out_ref[0, 0] = jnp.sum(d * d) * jnp.float32(1.0 / (NB * NF * 2))


@jax.jit
def _tc_combine(avg_t, prob_t, mask):
    return pl.pallas_call(
        _combine_body,
        out_specs=pl.BlockSpec(memory_space=pltpu.SMEM),
        out_shape=jax.ShapeDtypeStruct((1, 1), jnp.float32),
    )(avg_t, prob_t, mask)


def kernel(flow, prob, aggr_info):
    flow3 = (
        jnp.transpose(flow, (0, 2, 3, 4, 1))
        .reshape(NB, NG // 2, 2, NF // 128, 128)
        .transpose(0, 1, 3, 2, 4)
        .reshape(NB, 5120, 128)
    )
    prob_t = jnp.transpose(prob, (0, 2, 1))
    mask = _sc_mask(aggr_info)
    avg_t = _tc_reduce(flow3)
    loss = _tc_combine(avg_t, prob_t, mask)[0, 0]
    avg10 = jnp.transpose(avg_t[0, :, 0:10])
    return (loss, avg10)


# speedup vs baseline: 5.9747x; 5.9389x over previous
import functools

import jax
import jax.numpy as jnp
from jax import lax
from jax.experimental import pallas as pl
from jax.experimental.pallas import tpu as pltpu
from jax.experimental.pallas import tpu_sc as plsc

NB = 64
NF = 1280
NG = 512
NIDS = 65536
LANES = 16
NWORKERS = 32
BPW = NB // NWORKERS


def _sc_mask_body(aggr_hbm, mask_hbm, ids_v, mask_v):
    cid = lax.axis_index("c")
    sid = lax.axis_index("s")
    wid = sid * 2 + cid
    zeros = jnp.zeros((LANES,), jnp.float32)
    ones = jnp.ones((LANES,), jnp.float32)
    for bi in range(BPW):
        b = wid * BPW + bi

        def _zero(i, _):
            mask_v[pl.ds(i * LANES, LANES)] = zeros
            return 0

        lax.fori_loop(0, NF // LANES, _zero, 0, unroll=8)
        pltpu.sync_copy(aggr_hbm.at[b], ids_v)

        def _scatter(i, _):
            idx = ids_v[pl.ds(i * LANES, LANES)]
            plsc.store_scatter(mask_v, [idx], ones)
            return 0

        lax.fori_loop(0, NIDS // LANES, _scatter, 0, unroll=16)
        pltpu.sync_copy(mask_v, mask_hbm.at[b])


@jax.jit
def _sc_mask(aggr_info):
    mesh = plsc.VectorSubcoreMesh(core_axis_name="c", subcore_axis_name="s")
    return pl.kernel(
        _sc_mask_body,
        out_type=jax.ShapeDtypeStruct((NB, NF), jnp.float32),
        mesh=mesh,
        scratch_types=[
            pltpu.VMEM((NIDS,), jnp.int32),
            pltpu.VMEM((NF,), jnp.float32),
        ],
        compiler_params=pltpu.CompilerParams(needs_layout_passes=False),
    )(aggr_info)


def _reduce_body(flow_ref, out_ref):
    s = flow_ref[0, 0:128] + flow_ref[0, 128:256]
    for half in (64, 32, 16, 8, 4, 2, 1):
        s = s[0:half] + s[half : 2 * half]
    out_ref[0] = s[0] * jnp.float32(2.0 / NG)


@jax.jit
def _tc_reduce(flow4):
    return pl.pallas_call(
        _reduce_body,
        grid=(NB,),
        in_specs=[pl.BlockSpec((1, NG // 2, 2, NF), lambda b: (b, 0, 0, 0))],
        out_specs=pl.BlockSpec((1, 2, NF), lambda b: (b, 0, 0)),
        out_shape=jax.ShapeDtypeStruct((NB, 2, NF), jnp.float32),
    )(flow4)


def _combine_body(avg_ref, prob_ref, mask_ref, out_ref):
    d = (avg_ref[...] - prob_ref[...]) * mask_ref[...][:, None, :]
    out_ref[0, 0] = jnp.sum(d * d) * jnp.float32(1.0 / (NB * NF * 2))


@jax.jit
def _tc_combine(avg_t, prob_t, mask):
    return pl.pallas_call(
        _combine_body,
        out_specs=pl.BlockSpec(memory_space=pltpu.SMEM),
        out_shape=jax.ShapeDtypeStruct((1, 1), jnp.float32),
    )(avg_t, prob_t, mask)


def kernel(flow, prob, aggr_info):
    flow4 = jnp.transpose(flow, (0, 2, 3, 4, 1)).reshape(NB, NG // 2, 2, NF)
    prob_t = jnp.transpose(prob, (0, 2, 1))
    mask = _sc_mask(aggr_info)
    avg_t = _tc_reduce(flow4)
    loss = _tc_combine(avg_t, prob_t, mask)[0, 0]
    avg10 = jnp.transpose(avg_t[0, :, 0:10])
    return (loss, avg10)
